# Initial kernel scaffold; baseline (speedup 1.0000x reference)
import jax
import jax.numpy as jnp
from jax.experimental import pallas as pl


def kernel(inputs, edge_index, w, b, a):
    raise NotImplementedError("write your pallas kernel here")


# speedup vs baseline: 6.2838x; 6.2838x over previous
import functools

import jax
import jax.numpy as jnp
from jax import lax
from jax.experimental import pallas as pl
from jax.experimental.pallas import tpu as pltpu
from jax.experimental.pallas import tpu_sc as plsc

N = 10000
E = 320000
D = 128

NC, NS, L = 2, 16, 16
NW = NC * NS
CHUNK = 128
NCHUNKS = 79
EPW = CHUNK * NCHUNKS
E_PAD = NW * EPW
DUMMY = N
NLOC = 10240
ROWS_SH = 10240
RPT = ROWS_SH // NS

_f32 = jnp.float32


def _proj_body(x_ref, w_ref, a_ref, xw_ref, s_ref):
    x = x_ref[...]
    xw_ref[...] = jnp.dot(x, w_ref[...], preferred_element_type=_f32)
    s_ref[...] = jnp.dot(x, a_ref[...], preferred_element_type=_f32)


def _proj(x, w, a):
    blk = 1000
    return pl.pallas_call(
        _proj_body,
        grid=(N // blk,),
        in_specs=[
            pl.BlockSpec((blk, D), lambda i: (i, 0)),
            pl.BlockSpec((D, D), lambda i: (0, 0)),
            pl.BlockSpec((D, 1), lambda i: (0, 0)),
        ],
        out_specs=[
            pl.BlockSpec((blk, D), lambda i: (i, 0)),
            pl.BlockSpec((blk, 1), lambda i: (i, 0)),
        ],
        out_shape=[
            jax.ShapeDtypeStruct((N, D), _f32),
            jax.ShapeDtypeStruct((N, 1), _f32),
        ],
    )(x, w, a)


def _sc_body(src_hbm, dst_hbm, s_hbm, xw_hbm, acc_hbm, rs_hbm,
             s_loc, rs_loc, src_c, dst_c, we_c, rows, acc_sh, sem):
    c = lax.axis_index("c")
    t = lax.axis_index("s")
    wid = c * NS + t

    pltpu.sync_copy(s_hbm, s_loc.at[pl.ds(0, N)])
    s_loc[pl.ds(N, L)] = jnp.zeros((L,), _f32)

    def _zrs(i, carry):
        rs_loc[pl.ds(i * L, L)] = jnp.zeros((L,), _f32)
        return carry
    lax.fori_loop(0, NLOC // L, _zrs, 0)

    def _zrows(e, carry):
        for q in range(D // L):
            rows[e, pl.ds(q * L, L)] = jnp.zeros((L,), _f32)
        return carry
    lax.fori_loop(0, CHUNK, _zrows, 0)
    for k in range(RPT // CHUNK):
        pltpu.sync_copy(rows, acc_sh.at[pl.ds(t * RPT + k * CHUNK, CHUNK)])
    plsc.subcore_barrier()

    def _chunk(j, carry):
        off = wid * EPW + j * CHUNK
        pltpu.sync_copy(src_hbm.at[pl.ds(off, CHUNK)], src_c)
        pltpu.sync_copy(dst_hbm.at[pl.ds(off, CHUNK)], dst_c)

        def _we(i, carry2):
            sl = pl.ds(i * L, L)
            sv = src_c[sl]
            dv = dst_c[sl]
            z = plsc.load_gather(s_loc, [sv]) + plsc.load_gather(s_loc, [dv])
            z = jnp.where(z > 0.0, z, 0.2 * z)
            wv = jnp.exp(-z)
            we_c[sl] = wv
            plsc.addupdate_scatter(rs_loc, [sv], wv)
            return carry2
        lax.fori_loop(0, CHUNK // L, _we, 0)

        pltpu.async_copy(xw_hbm.at[dst_c], rows, sem).wait()

        def _scale(i, carry2):
            wvec = we_c[pl.ds(i * L, L)]
            for lane in range(L):
                wv = wvec[lane]
                e = i * L + lane
                for q in range(D // L):
                    sl = pl.ds(q * L, L)
                    rows[e, sl] = rows[e, sl] * wv
            return carry2
        lax.fori_loop(0, CHUNK // L, _scale, 0)

        pltpu.sync_copy(rows, acc_sh.at[src_c], add=True)
        return carry
    lax.fori_loop(0, NCHUNKS, _chunk, 0)

    pltpu.sync_copy(rs_loc, rs_hbm.at[pl.ds(wid * NLOC, NLOC)])

    plsc.subcore_barrier()
    for k in range(RPT // CHUNK):
        sl = pl.ds(t * RPT + k * CHUNK, CHUNK)
        pltpu.sync_copy(acc_sh.at[sl], rows)
        pltpu.sync_copy(rows, acc_hbm.at[c, sl])


def _sc_edges(src, dst, s, xw):
    mesh = plsc.VectorSubcoreMesh(core_axis_name="c", subcore_axis_name="s")
    return pl.kernel(
        _sc_body,
        out_type=[
            jax.ShapeDtypeStruct((NC, ROWS_SH, D), _f32),
            jax.ShapeDtypeStruct((NW * NLOC,), _f32),
        ],
        mesh=mesh,
        compiler_params=pltpu.CompilerParams(needs_layout_passes=False),
        scratch_types=[
            pltpu.VMEM((NLOC,), _f32),
            pltpu.VMEM((NLOC,), _f32),
            pltpu.VMEM((CHUNK,), jnp.int32),
            pltpu.VMEM((CHUNK,), jnp.int32),
            pltpu.VMEM((CHUNK,), _f32),
            pltpu.VMEM((CHUNK, D), _f32),
            pltpu.VMEM_SHARED((ROWS_SH, D), _f32),
            pltpu.SemaphoreType.DMA,
        ],
    )(src, dst, s, xw)


def _post_body(acc_ref, rs_ref, xw_ref, b_ref, o_ref):
    i = pl.program_id(0)
    rs = jnp.sum(rs_ref[:, pl.ds(i * 1024, 1024)], axis=0)
    rs = jnp.where(rs == 0.0, 1.0, rs)
    y = (acc_ref[0] + acc_ref[1]) / rs[:, None] + xw_ref[...] + b_ref[...]
    o_ref[...] = jnp.where(y > 0.0, y, 0.2 * y)


def _post(acc_p, rs_p, xw, b2):
    blk = 1024
    return pl.pallas_call(
        _post_body,
        grid=(pl.cdiv(N, blk),),
        in_specs=[
            pl.BlockSpec((NC, blk, D), lambda i: (0, i, 0)),
            pl.BlockSpec((NW, NLOC), lambda i: (0, 0)),
            pl.BlockSpec((blk, D), lambda i: (i, 0)),
            pl.BlockSpec((1, D), lambda i: (0, 0)),
        ],
        out_specs=pl.BlockSpec((blk, D), lambda i: (i, 0)),
        out_shape=jax.ShapeDtypeStruct((N, D), _f32),
    )(acc_p, rs_p, xw, b2)


def kernel(inputs, edge_index, w, b, a):
    xw, s = _proj(inputs, w, a)
    pad = E_PAD - E
    src = jnp.concatenate([edge_index[0], jnp.full((pad,), DUMMY, jnp.int32)])
    dst = jnp.concatenate([edge_index[1], jnp.zeros((pad,), jnp.int32)])
    acc_p, rs_flat = _sc_edges(src, dst, s.reshape(N), xw)
    return _post(acc_p, rs_flat.reshape(NW, NLOC), xw, b.reshape(1, D))
